# Initial kernel scaffold; baseline (speedup 1.0000x reference)
import jax
import jax.numpy as jnp
from jax.experimental import pallas as pl


def kernel(x_user, x_item, edge_index_ui, edge_index_iu, W_user, b_user, W_item, b_item, Wl_ui_0, Wr_ui_0, b_ui_0, Wl_iu_0, Wr_iu_0, b_iu_0, Wl_ui_1, Wr_ui_1, b_ui_1, Wl_iu_1, Wr_iu_1, b_iu_1):
    raise NotImplementedError("write your pallas kernel here")


# speedup vs baseline: 3.1984x; 3.1984x over previous
import functools

import jax
import jax.numpy as jnp
from jax import lax
from jax.experimental import pallas as pl
from jax.experimental.pallas import tpu as pltpu
from jax.experimental.pallas import tpu_sc as plsc

N_NODES = 50000
DF = 128
H = 64
HH = H // 2
E = 800000
EPAD = 800768
E_PER_TILE = EPAD // 16
CHUNK = 128
N_CHUNKS = E_PER_TILE // CHUNK
N_ACC = N_NODES + 48
ROWS_PER_TILE = N_ACC // 16
R_BLK = 400


def _segsum_body(ht, srcp, dstp, out, acc, zb, gi, di, rows, sem):
    c = lax.axis_index("c")
    s = lax.axis_index("s")

    zero16 = jnp.zeros((16,), jnp.float32)

    def zrow(i, _):
        zb[i, pl.ds(0, 16)] = zero16
        zb[i, pl.ds(16, 16)] = zero16
        return 0

    lax.fori_loop(0, CHUNK, zrow, 0)
    rbase = s * ROWS_PER_TILE
    for j in range(ROWS_PER_TILE // CHUNK):
        pltpu.sync_copy(zb, acc.at[pl.ds(rbase + j * CHUNK, CHUNK)])
    rem = ROWS_PER_TILE % CHUNK
    if rem:
        pltpu.sync_copy(zb.at[pl.ds(0, rem)],
                        acc.at[pl.ds(rbase + (ROWS_PER_TILE // CHUNK) * CHUNK, rem)])
    plsc.subcore_barrier()

    ebase = s * E_PER_TILE
    goff = c * N_NODES

    def chunk(t, _):
        off = ebase + t * CHUNK
        pltpu.sync_copy(srcp.at[pl.ds(off, CHUNK)], gi)
        pltpu.sync_copy(dstp.at[pl.ds(off, CHUNK)], di)
        for k in range(CHUNK // 16):
            sl = pl.ds(k * 16, 16)
            gi[sl] = gi[sl] + goff
        pltpu.async_copy(ht.at[gi], rows, sem).wait()
        pltpu.sync_copy(rows, acc.at[di], add=True)
        return 0

    lax.fori_loop(0, N_CHUNKS, chunk, 0)
    plsc.subcore_barrier()

    obase = c * N_ACC + rbase
    pltpu.sync_copy(acc.at[pl.ds(rbase, ROWS_PER_TILE)],
                    out.at[pl.ds(obase, ROWS_PER_TILE)])


def _segsum(ht, srcp, dstp):
    mesh = plsc.VectorSubcoreMesh(core_axis_name="c", subcore_axis_name="s")
    return pl.kernel(
        _segsum_body,
        mesh=mesh,
        compiler_params=pltpu.CompilerParams(use_tc_tiling_on_sc=False),
        out_type=jax.ShapeDtypeStruct((2 * N_ACC, HH), jnp.float32),
        scratch_types=[
            pltpu.VMEM_SHARED((N_ACC, HH), jnp.float32),
            pltpu.VMEM((CHUNK, HH), jnp.float32),
            pltpu.VMEM((CHUNK,), jnp.int32),
            pltpu.VMEM((CHUNK,), jnp.int32),
            pltpu.VMEM((CHUNK, HH), jnp.float32),
            pltpu.SemaphoreType.DMA,
        ],
    )(ht, srcp, dstp)


def _counts_body(dsts, out, acc, ones_b, zb, di, sem):
    c = lax.axis_index("c")
    s = lax.axis_index("s")

    zero16 = jnp.zeros((16,), jnp.float32)
    one16 = jnp.ones((16,), jnp.float32)

    def fill(i, _):
        zb[i, pl.ds(0, 16)] = zero16
        ones_b[i, pl.ds(0, 16)] = one16
        return 0

    lax.fori_loop(0, CHUNK, fill, 0)
    rbase = s * ROWS_PER_TILE
    for j in range(ROWS_PER_TILE // CHUNK):
        pltpu.sync_copy(zb, acc.at[pl.ds(rbase + j * CHUNK, CHUNK)])
    rem = ROWS_PER_TILE % CHUNK
    if rem:
        pltpu.sync_copy(zb.at[pl.ds(0, rem)],
                        acc.at[pl.ds(rbase + (ROWS_PER_TILE // CHUNK) * CHUNK, rem)])
    plsc.subcore_barrier()

    ebase = c * EPAD + s * E_PER_TILE

    def chunk(t, _):
        off = ebase + t * CHUNK
        pltpu.sync_copy(dsts.at[pl.ds(off, CHUNK)], di)
        pltpu.sync_copy(ones_b, acc.at[di], add=True)
        return 0

    lax.fori_loop(0, N_CHUNKS, chunk, 0)
    plsc.subcore_barrier()

    obase = c * N_ACC + rbase
    pltpu.sync_copy(acc.at[pl.ds(rbase, ROWS_PER_TILE)],
                    out.at[pl.ds(obase, ROWS_PER_TILE)])


def _counts(dsts):
    mesh = plsc.VectorSubcoreMesh(core_axis_name="c", subcore_axis_name="s")
    return pl.kernel(
        _counts_body,
        mesh=mesh,
        compiler_params=pltpu.CompilerParams(use_tc_tiling_on_sc=False),
        out_type=jax.ShapeDtypeStruct((2 * N_ACC, 16), jnp.float32),
        scratch_types=[
            pltpu.VMEM_SHARED((N_ACC, 16), jnp.float32),
            pltpu.VMEM((CHUNK, 16), jnp.float32),
            pltpu.VMEM((CHUNK, 16), jnp.float32),
            pltpu.VMEM((CHUNK,), jnp.int32),
            pltpu.SemaphoreType.DMA,
        ],
    )(dsts)


def _enc_body(x_ref, w_ref, b_ref, out_ref):
    x = x_ref[...]
    h = jnp.dot(x, w_ref[...], preferred_element_type=jnp.float32)
    r = jnp.maximum(h + b_ref[0], 0.0)
    out_ref[0] = r[:, :HH]
    out_ref[1] = r[:, HH:]


def _encode(x, w, b):
    return pl.pallas_call(
        _enc_body,
        grid=(N_NODES // R_BLK,),
        in_specs=[
            pl.BlockSpec((R_BLK, DF), lambda i: (i, 0)),
            pl.BlockSpec((DF, H), lambda i: (0, 0)),
            pl.BlockSpec((1, H), lambda i: (0, 0)),
        ],
        out_specs=pl.BlockSpec((2, R_BLK, HH), lambda i: (0, i, 0)),
        out_shape=jax.ShapeDtypeStruct((2, N_NODES, HH), jnp.float32),
    )(x, w, b.reshape(1, H))


def _conv_body(split, a0_ref, a1_ref, cnt_ref, x0_ref, x1_ref,
               wl_ref, wr_ref, b_ref, out_ref):
    rcp = 1.0 / jnp.maximum(cnt_ref[:, 0:1], 1.0)
    m0 = a0_ref[0] * rcp
    m1 = a1_ref[0] * rcp
    wl = wl_ref[...]
    wr = wr_ref[...]
    h = (jnp.dot(m0, wl[:HH], preferred_element_type=jnp.float32)
         + jnp.dot(m1, wl[HH:], preferred_element_type=jnp.float32)
         + jnp.dot(x0_ref[0], wr[:HH], preferred_element_type=jnp.float32)
         + jnp.dot(x1_ref[0], wr[HH:], preferred_element_type=jnp.float32))
    r = jnp.maximum(h + b_ref[0], 0.0)
    if split:
        out_ref[0] = r[:, :HH]
        out_ref[1] = r[:, HH:]
    else:
        out_ref[...] = r


def _conv_dense(agg, cnt, xd, wl, wr, b, split):
    if split:
        out_spec = pl.BlockSpec((2, R_BLK, HH), lambda i: (0, i, 0))
        out_shape = jax.ShapeDtypeStruct((2, N_NODES, HH), jnp.float32)
    else:
        out_spec = pl.BlockSpec((R_BLK, H), lambda i: (i, 0))
        out_shape = jax.ShapeDtypeStruct((N_NODES, H), jnp.float32)
    return pl.pallas_call(
        functools.partial(_conv_body, split),
        grid=(N_NODES // R_BLK,),
        in_specs=[
            pl.BlockSpec((1, R_BLK, HH), lambda i: (0, i, 0)),
            pl.BlockSpec((1, R_BLK, HH), lambda i: (1, i, 0)),
            pl.BlockSpec((R_BLK, 16), lambda i: (i, 0)),
            pl.BlockSpec((1, R_BLK, HH), lambda i: (0, i, 0)),
            pl.BlockSpec((1, R_BLK, HH), lambda i: (1, i, 0)),
            pl.BlockSpec((H, H), lambda i: (0, 0)),
            pl.BlockSpec((H, H), lambda i: (0, 0)),
            pl.BlockSpec((1, H), lambda i: (0, 0)),
        ],
        out_specs=out_spec,
        out_shape=out_shape,
    )(agg, agg, cnt, xd, xd, wl, wr, b.reshape(1, H))


def kernel(x_user, x_item, edge_index_ui, edge_index_iu,
           W_user, b_user, W_item, b_item,
           Wl_ui_0, Wr_ui_0, b_ui_0, Wl_iu_0, Wr_iu_0, b_iu_0,
           Wl_ui_1, Wr_ui_1, b_ui_1, Wl_iu_1, Wr_iu_1, b_iu_1):
    npad = EPAD - E
    pad_src = jnp.zeros((npad,), jnp.int32)
    pad_dst = (N_NODES + (jnp.arange(npad, dtype=jnp.int32) % 48))

    src_ui = jnp.concatenate([edge_index_ui[0], pad_src])
    dst_ui = jnp.concatenate([edge_index_ui[1], pad_dst])
    src_iu = jnp.concatenate([edge_index_iu[0], pad_src])
    dst_iu = jnp.concatenate([edge_index_iu[1], pad_dst])

    cnt2 = _counts(jnp.concatenate([dst_ui, dst_iu])).reshape(2, N_ACC, 16)
    cnt_ui = cnt2[0]
    cnt_iu = cnt2[1]

    xu = _encode(x_user, W_user, b_user)
    xi = _encode(x_item, W_item, b_item)

    layers = [(Wl_ui_0, Wr_ui_0, b_ui_0, Wl_iu_0, Wr_iu_0, b_iu_0, True),
              (Wl_ui_1, Wr_ui_1, b_ui_1, Wl_iu_1, Wr_iu_1, b_iu_1, False)]
    for (Wl_ui, Wr_ui, b_ui, Wl_iu, Wr_iu, b_iu, split) in layers:
        agg_i = _segsum(xu.reshape(2 * N_NODES, HH), src_ui, dst_ui)
        agg_u = _segsum(xi.reshape(2 * N_NODES, HH), src_iu, dst_iu)
        new_xi = _conv_dense(agg_i.reshape(2, N_ACC, HH), cnt_ui, xi,
                             Wl_ui, Wr_ui, b_ui, split)
        new_xu = _conv_dense(agg_u.reshape(2, N_ACC, HH), cnt_iu, xu,
                             Wl_iu, Wr_iu, b_iu, split)
        xu, xi = new_xu, new_xi
    return (xu, xi)


# speedup vs baseline: 6.5026x; 2.0331x over previous
import functools

import jax
import jax.numpy as jnp
from jax import lax
from jax.experimental import pallas as pl
from jax.experimental.pallas import tpu as pltpu
from jax.experimental.pallas import tpu_sc as plsc

N_NODES = 50000
DF = 128
H = 64
HH = H // 2
E = 800000
EPAD = 802816
E_PER_TILE = EPAD // 16
CHUNK = 128
N_CHUNKS = E_PER_TILE // CHUNK
GRP = 4
N_GRPS = N_CHUNKS // GRP
N_ACC = N_NODES + 48
ROWS_PER_TILE = N_ACC // 16
R_BLK = 400


def _segsum_body(ht, srcp, dstp, out, acc, zb, gi, di,
                 r0, r1, r2, r3, g0, g1, g2, g3, s0, s1, s2, s3):
    rows = [r0, r1, r2, r3]
    gsem = [g0, g1, g2, g3]
    ssem = [s0, s1, s2, s3]
    c = lax.axis_index("c")
    s = lax.axis_index("s")

    zero16 = jnp.zeros((16,), jnp.float32)

    def zrow(i, _):
        zb[i, pl.ds(0, 16)] = zero16
        zb[i, pl.ds(16, 16)] = zero16
        return 0

    lax.fori_loop(0, CHUNK, zrow, 0)
    rbase = s * ROWS_PER_TILE
    for j in range(ROWS_PER_TILE // CHUNK):
        pltpu.sync_copy(zb, acc.at[pl.ds(rbase + j * CHUNK, CHUNK)])
    rem = ROWS_PER_TILE % CHUNK
    if rem:
        pltpu.sync_copy(zb.at[pl.ds(0, rem)],
                        acc.at[pl.ds(rbase + (ROWS_PER_TILE // CHUNK) * CHUNK, rem)])
    plsc.subcore_barrier()

    crow_base = s * N_CHUNKS
    goff = c * N_NODES

    def group(t, _):
        crow = crow_base + t * GRP
        pltpu.sync_copy(srcp.at[pl.ds(crow, GRP)], gi)
        pltpu.sync_copy(dstp.at[pl.ds(crow, GRP)], di)
        for j in range(GRP):
            for k in range(CHUNK // 16):
                sl = pl.ds(k * 16, 16)
                gi[j, sl] = gi[j, sl] + goff
        gd = [pltpu.async_copy(ht.at[gi.at[j]], rows[j], gsem[j])
              for j in range(GRP)]
        sd = []
        for j in range(GRP):
            gd[j].wait()
            sd.append(pltpu.async_copy(rows[j], acc.at[di.at[j]],
                                       ssem[j], add=True))
        for j in range(GRP):
            sd[j].wait()
        return 0

    lax.fori_loop(0, N_GRPS, group, 0)
    plsc.subcore_barrier()

    obase = c * N_ACC + rbase
    pltpu.sync_copy(acc.at[pl.ds(rbase, ROWS_PER_TILE)],
                    out.at[pl.ds(obase, ROWS_PER_TILE)])


def _segsum(ht, srcp, dstp):
    mesh = plsc.VectorSubcoreMesh(core_axis_name="c", subcore_axis_name="s")
    return pl.kernel(
        _segsum_body,
        mesh=mesh,
        compiler_params=pltpu.CompilerParams(use_tc_tiling_on_sc=False),
        out_type=jax.ShapeDtypeStruct((2 * N_ACC, HH), jnp.float32),
        scratch_types=[
            pltpu.VMEM_SHARED((N_ACC, HH), jnp.float32),
            pltpu.VMEM((CHUNK, HH), jnp.float32),
            pltpu.VMEM((GRP, CHUNK), jnp.int32),
            pltpu.VMEM((GRP, CHUNK), jnp.int32),
        ] + [pltpu.VMEM((CHUNK, HH), jnp.float32) for _ in range(GRP)]
          + [pltpu.SemaphoreType.DMA for _ in range(2 * GRP)],
    )(ht.reshape(2 * N_NODES, HH),
      srcp.reshape(EPAD // CHUNK, CHUNK),
      dstp.reshape(EPAD // CHUNK, CHUNK))


def _counts_body(dsts, out, acc, ones_b, zb, di, s0, s1, s2, s3):
    ssem = [s0, s1, s2, s3]
    c = lax.axis_index("c")
    s = lax.axis_index("s")

    zero16 = jnp.zeros((16,), jnp.float32)
    one16 = jnp.ones((16,), jnp.float32)

    def fill(i, _):
        zb[i, pl.ds(0, 16)] = zero16
        ones_b[i, pl.ds(0, 16)] = one16
        return 0

    lax.fori_loop(0, CHUNK, fill, 0)
    rbase = s * ROWS_PER_TILE
    for j in range(ROWS_PER_TILE // CHUNK):
        pltpu.sync_copy(zb, acc.at[pl.ds(rbase + j * CHUNK, CHUNK)])
    rem = ROWS_PER_TILE % CHUNK
    if rem:
        pltpu.sync_copy(zb.at[pl.ds(0, rem)],
                        acc.at[pl.ds(rbase + (ROWS_PER_TILE // CHUNK) * CHUNK, rem)])
    plsc.subcore_barrier()

    crow_base = (c * 16 + s) * N_CHUNKS

    def group(t, _):
        crow = crow_base + t * GRP
        pltpu.sync_copy(dsts.at[pl.ds(crow, GRP)], di)
        sd = [pltpu.async_copy(ones_b, acc.at[di.at[j]], ssem[j], add=True)
              for j in range(GRP)]
        for j in range(GRP):
            sd[j].wait()
        return 0

    lax.fori_loop(0, N_GRPS, group, 0)
    plsc.subcore_barrier()

    obase = c * N_ACC + rbase
    pltpu.sync_copy(acc.at[pl.ds(rbase, ROWS_PER_TILE)],
                    out.at[pl.ds(obase, ROWS_PER_TILE)])


def _counts(dsts):
    mesh = plsc.VectorSubcoreMesh(core_axis_name="c", subcore_axis_name="s")
    return pl.kernel(
        _counts_body,
        mesh=mesh,
        compiler_params=pltpu.CompilerParams(use_tc_tiling_on_sc=False),
        out_type=jax.ShapeDtypeStruct((2 * N_ACC, 16), jnp.float32),
        scratch_types=[
            pltpu.VMEM_SHARED((N_ACC, 16), jnp.float32),
            pltpu.VMEM((CHUNK, 16), jnp.float32),
            pltpu.VMEM((CHUNK, 16), jnp.float32),
            pltpu.VMEM((GRP, CHUNK), jnp.int32),
        ] + [pltpu.SemaphoreType.DMA for _ in range(GRP)],
    )(dsts.reshape(2 * EPAD // CHUNK, CHUNK))


def _enc_body(x_ref, w_ref, b_ref, out_ref):
    x = x_ref[...]
    h = jnp.dot(x, w_ref[...], preferred_element_type=jnp.float32)
    r = jnp.maximum(h + b_ref[0], 0.0)
    out_ref[0] = r[:, :HH]
    out_ref[1] = r[:, HH:]


def _encode(x, w, b):
    return pl.pallas_call(
        _enc_body,
        grid=(N_NODES // R_BLK,),
        in_specs=[
            pl.BlockSpec((R_BLK, DF), lambda i: (i, 0)),
            pl.BlockSpec((DF, H), lambda i: (0, 0)),
            pl.BlockSpec((1, H), lambda i: (0, 0)),
        ],
        out_specs=pl.BlockSpec((2, R_BLK, HH), lambda i: (0, i, 0)),
        out_shape=jax.ShapeDtypeStruct((2, N_NODES, HH), jnp.float32),
    )(x, w, b.reshape(1, H))


def _conv_body(split, a0_ref, a1_ref, cnt_ref, x0_ref, x1_ref,
               wl_ref, wr_ref, b_ref, out_ref):
    rcp = 1.0 / jnp.maximum(cnt_ref[:, 0:1], 1.0)
    m0 = a0_ref[0] * rcp
    m1 = a1_ref[0] * rcp
    wl = wl_ref[...]
    wr = wr_ref[...]
    h = (jnp.dot(m0, wl[:HH], preferred_element_type=jnp.float32)
         + jnp.dot(m1, wl[HH:], preferred_element_type=jnp.float32)
         + jnp.dot(x0_ref[0], wr[:HH], preferred_element_type=jnp.float32)
         + jnp.dot(x1_ref[0], wr[HH:], preferred_element_type=jnp.float32))
    r = jnp.maximum(h + b_ref[0], 0.0)
    if split:
        out_ref[0] = r[:, :HH]
        out_ref[1] = r[:, HH:]
    else:
        out_ref[...] = r


def _conv_dense(agg, cnt, xd, wl, wr, b, split):
    if split:
        out_spec = pl.BlockSpec((2, R_BLK, HH), lambda i: (0, i, 0))
        out_shape = jax.ShapeDtypeStruct((2, N_NODES, HH), jnp.float32)
    else:
        out_spec = pl.BlockSpec((R_BLK, H), lambda i: (i, 0))
        out_shape = jax.ShapeDtypeStruct((N_NODES, H), jnp.float32)
    return pl.pallas_call(
        functools.partial(_conv_body, split),
        grid=(N_NODES // R_BLK,),
        in_specs=[
            pl.BlockSpec((1, R_BLK, HH), lambda i: (0, i, 0)),
            pl.BlockSpec((1, R_BLK, HH), lambda i: (1, i, 0)),
            pl.BlockSpec((R_BLK, 16), lambda i: (i, 0)),
            pl.BlockSpec((1, R_BLK, HH), lambda i: (0, i, 0)),
            pl.BlockSpec((1, R_BLK, HH), lambda i: (1, i, 0)),
            pl.BlockSpec((H, H), lambda i: (0, 0)),
            pl.BlockSpec((H, H), lambda i: (0, 0)),
            pl.BlockSpec((1, H), lambda i: (0, 0)),
        ],
        out_specs=out_spec,
        out_shape=out_shape,
    )(agg, agg, cnt, xd, xd, wl, wr, b.reshape(1, H))


def kernel(x_user, x_item, edge_index_ui, edge_index_iu,
           W_user, b_user, W_item, b_item,
           Wl_ui_0, Wr_ui_0, b_ui_0, Wl_iu_0, Wr_iu_0, b_iu_0,
           Wl_ui_1, Wr_ui_1, b_ui_1, Wl_iu_1, Wr_iu_1, b_iu_1):
    npad = EPAD - E
    pad_src = jnp.zeros((npad,), jnp.int32)
    pad_dst = (N_NODES + (jnp.arange(npad, dtype=jnp.int32) % 48))

    src_ui = jnp.concatenate([edge_index_ui[0], pad_src])
    dst_ui = jnp.concatenate([edge_index_ui[1], pad_dst])
    src_iu = jnp.concatenate([edge_index_iu[0], pad_src])
    dst_iu = jnp.concatenate([edge_index_iu[1], pad_dst])

    cnt2 = _counts(jnp.concatenate([dst_ui, dst_iu])).reshape(2, N_ACC, 16)
    cnt_ui = cnt2[0]
    cnt_iu = cnt2[1]

    xu = _encode(x_user, W_user, b_user)
    xi = _encode(x_item, W_item, b_item)

    layers = [(Wl_ui_0, Wr_ui_0, b_ui_0, Wl_iu_0, Wr_iu_0, b_iu_0, True),
              (Wl_ui_1, Wr_ui_1, b_ui_1, Wl_iu_1, Wr_iu_1, b_iu_1, False)]
    for (Wl_ui, Wr_ui, b_ui, Wl_iu, Wr_iu, b_iu, split) in layers:
        agg_i = _segsum(xu.reshape(2 * N_NODES, HH), src_ui, dst_ui)
        agg_u = _segsum(xi.reshape(2 * N_NODES, HH), src_iu, dst_iu)
        new_xi = _conv_dense(agg_i.reshape(2, N_ACC, HH), cnt_ui, xi,
                             Wl_ui, Wr_ui, b_ui, split)
        new_xu = _conv_dense(agg_u.reshape(2, N_ACC, HH), cnt_iu, xu,
                             Wl_iu, Wr_iu, b_iu, split)
        xu, xi = new_xu, new_xi
    return (xu, xi)


# speedup vs baseline: 7.5989x; 1.1686x over previous
import functools

import jax
import jax.numpy as jnp
from jax import lax
from jax.experimental import pallas as pl
from jax.experimental.pallas import tpu as pltpu
from jax.experimental.pallas import tpu_sc as plsc

N_NODES = 50000
DF = 128
H = 64
HH = H // 2
E = 800000
EPAD = 802816
E_PER_TILE = EPAD // 16
CHUNK = 128
N_CHUNKS = E_PER_TILE // CHUNK
GRP = 7
N_GRPS = N_CHUNKS // GRP
N_ACC = N_NODES + 48
ROWS_PER_TILE = N_ACC // 16
R_BLK = 400


def _segsum_body(ht, srcp, dstp, out, acc, gi, di, *bufs):
    rows = bufs[:GRP]
    gsem = bufs[GRP:2 * GRP]
    ssem = bufs[2 * GRP:3 * GRP]
    c = lax.axis_index("c")
    s = lax.axis_index("s")

    zb = rows[0]
    zero16 = jnp.zeros((16,), jnp.float32)

    def zrow(i, _):
        zb[i, pl.ds(0, 16)] = zero16
        zb[i, pl.ds(16, 16)] = zero16
        return 0

    lax.fori_loop(0, CHUNK, zrow, 0)
    rbase = s * ROWS_PER_TILE
    for j in range(ROWS_PER_TILE // CHUNK):
        pltpu.sync_copy(zb, acc.at[pl.ds(rbase + j * CHUNK, CHUNK)])
    rem = ROWS_PER_TILE % CHUNK
    if rem:
        pltpu.sync_copy(zb.at[pl.ds(0, rem)],
                        acc.at[pl.ds(rbase + (ROWS_PER_TILE // CHUNK) * CHUNK, rem)])
    plsc.subcore_barrier()

    crow_base = s * N_CHUNKS
    goff = c * N_NODES

    def group(t, _):
        crow = crow_base + t * GRP
        pltpu.sync_copy(srcp.at[pl.ds(crow, GRP)], gi)
        pltpu.sync_copy(dstp.at[pl.ds(crow, GRP)], di)
        for j in range(GRP):
            for k in range(CHUNK // 16):
                sl = pl.ds(k * 16, 16)
                gi[j, sl] = gi[j, sl] + goff
        gd = [pltpu.async_copy(ht.at[gi.at[j]], rows[j], gsem[j])
              for j in range(GRP)]
        sd = []
        for j in range(GRP):
            gd[j].wait()
            sd.append(pltpu.async_copy(rows[j], acc.at[di.at[j]],
                                       ssem[j], add=True))
        for j in range(GRP):
            sd[j].wait()
        return 0

    lax.fori_loop(0, N_GRPS, group, 0)
    plsc.subcore_barrier()

    obase = c * N_ACC + rbase
    pltpu.sync_copy(acc.at[pl.ds(rbase, ROWS_PER_TILE)],
                    out.at[pl.ds(obase, ROWS_PER_TILE)])


def _segsum(ht, srcp, dstp):
    mesh = plsc.VectorSubcoreMesh(core_axis_name="c", subcore_axis_name="s")
    return pl.kernel(
        _segsum_body,
        mesh=mesh,
        compiler_params=pltpu.CompilerParams(use_tc_tiling_on_sc=False),
        out_type=jax.ShapeDtypeStruct((2 * N_ACC, HH), jnp.float32),
        scratch_types=[
            pltpu.VMEM_SHARED((N_ACC, HH), jnp.float32),
            pltpu.VMEM((GRP, CHUNK), jnp.int32),
            pltpu.VMEM((GRP, CHUNK), jnp.int32),
        ] + [pltpu.VMEM((CHUNK, HH), jnp.float32) for _ in range(GRP)]
          + [pltpu.SemaphoreType.DMA for _ in range(2 * GRP)],
    )(ht.reshape(2 * N_NODES, HH),
      srcp.reshape(EPAD // CHUNK, CHUNK),
      dstp.reshape(EPAD // CHUNK, CHUNK))


def _counts_body(dsts, out, acc, ones_b, zb, di, *ssem):
    c = lax.axis_index("c")
    s = lax.axis_index("s")

    zero16 = jnp.zeros((16,), jnp.float32)
    one16 = jnp.ones((16,), jnp.float32)

    def fill(i, _):
        zb[i, pl.ds(0, 16)] = zero16
        ones_b[i, pl.ds(0, 16)] = one16
        return 0

    lax.fori_loop(0, CHUNK, fill, 0)
    rbase = s * ROWS_PER_TILE
    for j in range(ROWS_PER_TILE // CHUNK):
        pltpu.sync_copy(zb, acc.at[pl.ds(rbase + j * CHUNK, CHUNK)])
    rem = ROWS_PER_TILE % CHUNK
    if rem:
        pltpu.sync_copy(zb.at[pl.ds(0, rem)],
                        acc.at[pl.ds(rbase + (ROWS_PER_TILE // CHUNK) * CHUNK, rem)])
    plsc.subcore_barrier()

    crow_base = (c * 16 + s) * N_CHUNKS

    def group(t, _):
        crow = crow_base + t * GRP
        pltpu.sync_copy(dsts.at[pl.ds(crow, GRP)], di)
        sd = [pltpu.async_copy(ones_b, acc.at[di.at[j]], ssem[j], add=True)
              for j in range(GRP)]
        for j in range(GRP):
            sd[j].wait()
        return 0

    lax.fori_loop(0, N_GRPS, group, 0)
    plsc.subcore_barrier()

    obase = c * N_ACC + rbase
    pltpu.sync_copy(acc.at[pl.ds(rbase, ROWS_PER_TILE)],
                    out.at[pl.ds(obase, ROWS_PER_TILE)])


def _counts(dsts):
    mesh = plsc.VectorSubcoreMesh(core_axis_name="c", subcore_axis_name="s")
    return pl.kernel(
        _counts_body,
        mesh=mesh,
        compiler_params=pltpu.CompilerParams(use_tc_tiling_on_sc=False),
        out_type=jax.ShapeDtypeStruct((2 * N_ACC, 16), jnp.float32),
        scratch_types=[
            pltpu.VMEM_SHARED((N_ACC, 16), jnp.float32),
            pltpu.VMEM((CHUNK, 16), jnp.float32),
            pltpu.VMEM((CHUNK, 16), jnp.float32),
            pltpu.VMEM((GRP, CHUNK), jnp.int32),
        ] + [pltpu.SemaphoreType.DMA for _ in range(GRP)],
    )(dsts.reshape(2 * EPAD // CHUNK, CHUNK))


def _enc_body(x_ref, w_ref, b_ref, out_ref):
    x = x_ref[...]
    h = jnp.dot(x, w_ref[...], preferred_element_type=jnp.float32)
    r = jnp.maximum(h + b_ref[0], 0.0)
    out_ref[0] = r[:, :HH]
    out_ref[1] = r[:, HH:]


def _encode(x, w, b):
    return pl.pallas_call(
        _enc_body,
        grid=(N_NODES // R_BLK,),
        in_specs=[
            pl.BlockSpec((R_BLK, DF), lambda i: (i, 0)),
            pl.BlockSpec((DF, H), lambda i: (0, 0)),
            pl.BlockSpec((1, H), lambda i: (0, 0)),
        ],
        out_specs=pl.BlockSpec((2, R_BLK, HH), lambda i: (0, i, 0)),
        out_shape=jax.ShapeDtypeStruct((2, N_NODES, HH), jnp.float32),
    )(x, w, b.reshape(1, H))


def _conv_body(split, a0_ref, a1_ref, cnt_ref, x0_ref, x1_ref,
               wl_ref, wr_ref, b_ref, out_ref):
    rcp = 1.0 / jnp.maximum(cnt_ref[:, 0:1], 1.0)
    m0 = a0_ref[0] * rcp
    m1 = a1_ref[0] * rcp
    wl = wl_ref[...]
    wr = wr_ref[...]
    h = (jnp.dot(m0, wl[:HH], preferred_element_type=jnp.float32)
         + jnp.dot(m1, wl[HH:], preferred_element_type=jnp.float32)
         + jnp.dot(x0_ref[0], wr[:HH], preferred_element_type=jnp.float32)
         + jnp.dot(x1_ref[0], wr[HH:], preferred_element_type=jnp.float32))
    r = jnp.maximum(h + b_ref[0], 0.0)
    if split:
        out_ref[0] = r[:, :HH]
        out_ref[1] = r[:, HH:]
    else:
        out_ref[...] = r


def _conv_dense(agg, cnt, xd, wl, wr, b, split):
    if split:
        out_spec = pl.BlockSpec((2, R_BLK, HH), lambda i: (0, i, 0))
        out_shape = jax.ShapeDtypeStruct((2, N_NODES, HH), jnp.float32)
    else:
        out_spec = pl.BlockSpec((R_BLK, H), lambda i: (i, 0))
        out_shape = jax.ShapeDtypeStruct((N_NODES, H), jnp.float32)
    return pl.pallas_call(
        functools.partial(_conv_body, split),
        grid=(N_NODES // R_BLK,),
        in_specs=[
            pl.BlockSpec((1, R_BLK, HH), lambda i: (0, i, 0)),
            pl.BlockSpec((1, R_BLK, HH), lambda i: (1, i, 0)),
            pl.BlockSpec((R_BLK, 16), lambda i: (i, 0)),
            pl.BlockSpec((1, R_BLK, HH), lambda i: (0, i, 0)),
            pl.BlockSpec((1, R_BLK, HH), lambda i: (1, i, 0)),
            pl.BlockSpec((H, H), lambda i: (0, 0)),
            pl.BlockSpec((H, H), lambda i: (0, 0)),
            pl.BlockSpec((1, H), lambda i: (0, 0)),
        ],
        out_specs=out_spec,
        out_shape=out_shape,
    )(agg, agg, cnt, xd, xd, wl, wr, b.reshape(1, H))


def kernel(x_user, x_item, edge_index_ui, edge_index_iu,
           W_user, b_user, W_item, b_item,
           Wl_ui_0, Wr_ui_0, b_ui_0, Wl_iu_0, Wr_iu_0, b_iu_0,
           Wl_ui_1, Wr_ui_1, b_ui_1, Wl_iu_1, Wr_iu_1, b_iu_1):
    npad = EPAD - E
    pad_src = jnp.zeros((npad,), jnp.int32)
    pad_dst = (N_NODES + (jnp.arange(npad, dtype=jnp.int32) % 48))

    src_ui = jnp.concatenate([edge_index_ui[0], pad_src])
    dst_ui = jnp.concatenate([edge_index_ui[1], pad_dst])
    src_iu = jnp.concatenate([edge_index_iu[0], pad_src])
    dst_iu = jnp.concatenate([edge_index_iu[1], pad_dst])

    cnt2 = _counts(jnp.concatenate([dst_ui, dst_iu])).reshape(2, N_ACC, 16)
    cnt_ui = cnt2[0]
    cnt_iu = cnt2[1]

    xu = _encode(x_user, W_user, b_user)
    xi = _encode(x_item, W_item, b_item)

    layers = [(Wl_ui_0, Wr_ui_0, b_ui_0, Wl_iu_0, Wr_iu_0, b_iu_0, True),
              (Wl_ui_1, Wr_ui_1, b_ui_1, Wl_iu_1, Wr_iu_1, b_iu_1, False)]
    for (Wl_ui, Wr_ui, b_ui, Wl_iu, Wr_iu, b_iu, split) in layers:
        agg_i = _segsum(xu.reshape(2 * N_NODES, HH), src_ui, dst_ui)
        agg_u = _segsum(xi.reshape(2 * N_NODES, HH), src_iu, dst_iu)
        new_xi = _conv_dense(agg_i.reshape(2, N_ACC, HH), cnt_ui, xi,
                             Wl_ui, Wr_ui, b_ui, split)
        new_xu = _conv_dense(agg_u.reshape(2, N_ACC, HH), cnt_iu, xu,
                             Wl_iu, Wr_iu, b_iu, split)
        xu, xi = new_xu, new_xi
    return (xu, xi)


# speedup vs baseline: 7.8440x; 1.0323x over previous
import functools

import jax
import jax.numpy as jnp
from jax import lax
from jax.experimental import pallas as pl
from jax.experimental.pallas import tpu as pltpu
from jax.experimental.pallas import tpu_sc as plsc

N_NODES = 50000
DF = 128
H = 64
HH = H // 2
E = 800000
EPAD = 802816
E_PER_TILE = EPAD // 16
CHUNK = 128
N_CHUNKS = E_PER_TILE // CHUNK
GRP = 7
N_GRPS = N_CHUNKS // GRP
N_ACC = N_NODES + 48
ROWS_PER_TILE = N_ACC // 16
R_BLK = 1000


def _segsum_body(ht, srcp, dstp, out, acc, gi, di, *bufs):
    rows = bufs[:GRP]
    gsem = bufs[GRP:2 * GRP]
    ssem = bufs[2 * GRP:3 * GRP]
    isem = bufs[3 * GRP]
    c = lax.axis_index("c")
    s = lax.axis_index("s")

    zb = rows[0]
    zero16 = jnp.zeros((16,), jnp.float32)

    def zrow(i, _):
        zb[i, pl.ds(0, 16)] = zero16
        zb[i, pl.ds(16, 16)] = zero16
        return 0

    lax.fori_loop(0, CHUNK, zrow, 0)
    rbase = s * ROWS_PER_TILE
    for j in range(ROWS_PER_TILE // CHUNK):
        pltpu.sync_copy(zb, acc.at[pl.ds(rbase + j * CHUNK, CHUNK)])
    rem = ROWS_PER_TILE % CHUNK
    if rem:
        pltpu.sync_copy(zb.at[pl.ds(0, rem)],
                        acc.at[pl.ds(rbase + (ROWS_PER_TILE // CHUNK) * CHUNK, rem)])
    plsc.subcore_barrier()

    crow_base = s * N_CHUNKS
    goff = c * N_NODES

    def group(t, _):
        crow = crow_base + t * GRP
        gd_src = pltpu.async_copy(srcp.at[pl.ds(crow, GRP)], gi, isem)
        gd_dst = pltpu.async_copy(dstp.at[pl.ds(crow, GRP)], di, isem)
        gd_src.wait()
        gd_dst.wait()
        for j in range(GRP):
            for k in range(CHUNK // 16):
                sl = pl.ds(k * 16, 16)
                gi[j, sl] = gi[j, sl] + goff
        gd = [pltpu.async_copy(ht.at[gi.at[j]], rows[j], gsem[j])
              for j in range(GRP)]
        sd = []
        for j in range(GRP):
            gd[j].wait()
            sd.append(pltpu.async_copy(rows[j], acc.at[di.at[j]],
                                       ssem[j], add=True))
        for j in range(GRP):
            sd[j].wait()
        return 0

    lax.fori_loop(0, N_GRPS, group, 0)
    plsc.subcore_barrier()

    obase = c * N_ACC + rbase
    pltpu.sync_copy(acc.at[pl.ds(rbase, ROWS_PER_TILE)],
                    out.at[pl.ds(obase, ROWS_PER_TILE)])


def _segsum(ht, srcp, dstp):
    mesh = plsc.VectorSubcoreMesh(core_axis_name="c", subcore_axis_name="s")
    return pl.kernel(
        _segsum_body,
        mesh=mesh,
        compiler_params=pltpu.CompilerParams(use_tc_tiling_on_sc=False),
        out_type=jax.ShapeDtypeStruct((2 * N_ACC, HH), jnp.float32),
        scratch_types=[
            pltpu.VMEM_SHARED((N_ACC, HH), jnp.float32),
            pltpu.VMEM((GRP, CHUNK), jnp.int32),
            pltpu.VMEM((GRP, CHUNK), jnp.int32),
        ] + [pltpu.VMEM((CHUNK, HH), jnp.float32) for _ in range(GRP)]
          + [pltpu.SemaphoreType.DMA for _ in range(2 * GRP + 1)],
    )(ht.reshape(2 * N_NODES, HH),
      srcp.reshape(EPAD // CHUNK, CHUNK),
      dstp.reshape(EPAD // CHUNK, CHUNK))


def _counts_body(dsts, out, acc, ones_b, zb, di, *ssem):
    c = lax.axis_index("c")
    s = lax.axis_index("s")

    zero16 = jnp.zeros((16,), jnp.float32)
    one16 = jnp.ones((16,), jnp.float32)

    def fill(i, _):
        zb[i, pl.ds(0, 16)] = zero16
        ones_b[i, pl.ds(0, 16)] = one16
        return 0

    lax.fori_loop(0, CHUNK, fill, 0)
    rbase = s * ROWS_PER_TILE
    for j in range(ROWS_PER_TILE // CHUNK):
        pltpu.sync_copy(zb, acc.at[pl.ds(rbase + j * CHUNK, CHUNK)])
    rem = ROWS_PER_TILE % CHUNK
    if rem:
        pltpu.sync_copy(zb.at[pl.ds(0, rem)],
                        acc.at[pl.ds(rbase + (ROWS_PER_TILE // CHUNK) * CHUNK, rem)])
    plsc.subcore_barrier()

    crow_base = (c * 16 + s) * N_CHUNKS

    def group(t, _):
        crow = crow_base + t * GRP
        pltpu.sync_copy(dsts.at[pl.ds(crow, GRP)], di)
        sd = [pltpu.async_copy(ones_b, acc.at[di.at[j]], ssem[j], add=True)
              for j in range(GRP)]
        for j in range(GRP):
            sd[j].wait()
        return 0

    lax.fori_loop(0, N_GRPS, group, 0)
    plsc.subcore_barrier()

    obase = c * N_ACC + rbase
    pltpu.sync_copy(acc.at[pl.ds(rbase, ROWS_PER_TILE)],
                    out.at[pl.ds(obase, ROWS_PER_TILE)])


def _counts(dsts):
    mesh = plsc.VectorSubcoreMesh(core_axis_name="c", subcore_axis_name="s")
    return pl.kernel(
        _counts_body,
        mesh=mesh,
        compiler_params=pltpu.CompilerParams(use_tc_tiling_on_sc=False),
        out_type=jax.ShapeDtypeStruct((2 * N_ACC, 16), jnp.float32),
        scratch_types=[
            pltpu.VMEM_SHARED((N_ACC, 16), jnp.float32),
            pltpu.VMEM((CHUNK, 16), jnp.float32),
            pltpu.VMEM((CHUNK, 16), jnp.float32),
            pltpu.VMEM((GRP, CHUNK), jnp.int32),
        ] + [pltpu.SemaphoreType.DMA for _ in range(GRP)],
    )(dsts.reshape(2 * EPAD // CHUNK, CHUNK))


def _enc_body(xu_ref, wu_ref, bu_ref, xi_ref, wi_ref, bi_ref,
              ou_ref, oi_ref):
    for x_ref, w_ref, b_ref, out_ref in ((xu_ref, wu_ref, bu_ref, ou_ref),
                                         (xi_ref, wi_ref, bi_ref, oi_ref)):
        h = jnp.dot(x_ref[...], w_ref[...], preferred_element_type=jnp.float32)
        r = jnp.maximum(h + b_ref[0], 0.0)
        out_ref[0] = r[:, :HH]
        out_ref[1] = r[:, HH:]


def _encode2(xu, wu, bu, xi, wi, bi):
    return pl.pallas_call(
        _enc_body,
        grid=(N_NODES // R_BLK,),
        in_specs=[
            pl.BlockSpec((R_BLK, DF), lambda i: (i, 0)),
            pl.BlockSpec((DF, H), lambda i: (0, 0)),
            pl.BlockSpec((1, H), lambda i: (0, 0)),
            pl.BlockSpec((R_BLK, DF), lambda i: (i, 0)),
            pl.BlockSpec((DF, H), lambda i: (0, 0)),
            pl.BlockSpec((1, H), lambda i: (0, 0)),
        ],
        out_specs=[pl.BlockSpec((2, R_BLK, HH), lambda i: (0, i, 0))] * 2,
        out_shape=[jax.ShapeDtypeStruct((2, N_NODES, HH), jnp.float32)] * 2,
    )(xu, wu, bu.reshape(1, H), xi, wi, bi.reshape(1, H))


def _one_conv(split, a0_ref, a1_ref, cnt_ref, x0_ref, x1_ref,
              wl_ref, wr_ref, b_ref, out_ref):
    rcp = 1.0 / jnp.maximum(cnt_ref[:, 0:1], 1.0)
    m0 = a0_ref[0] * rcp
    m1 = a1_ref[0] * rcp
    wl = wl_ref[...]
    wr = wr_ref[...]
    h = (jnp.dot(m0, wl[:HH], preferred_element_type=jnp.float32)
         + jnp.dot(m1, wl[HH:], preferred_element_type=jnp.float32)
         + jnp.dot(x0_ref[0], wr[:HH], preferred_element_type=jnp.float32)
         + jnp.dot(x1_ref[0], wr[HH:], preferred_element_type=jnp.float32))
    r = jnp.maximum(h + b_ref[0], 0.0)
    if split:
        out_ref[0] = r[:, :HH]
        out_ref[1] = r[:, HH:]
    else:
        out_ref[...] = r


def _layer_body(split, ai0, ai1, ci, xi0, xi1, wli, wri, bi,
                au0, au1, cu, xu0, xu1, wlu, wru, bu, oi, ou):
    _one_conv(split, ai0, ai1, ci, xi0, xi1, wli, wri, bi, oi)
    _one_conv(split, au0, au1, cu, xu0, xu1, wlu, wru, bu, ou)


def _layer_dense(agg_i, cnt_ui, xi, wl_ui, wr_ui, b_ui,
                 agg_u, cnt_iu, xu, wl_iu, wr_iu, b_iu, split):
    if split:
        out_spec = pl.BlockSpec((2, R_BLK, HH), lambda i: (0, i, 0))
        out_shape = jax.ShapeDtypeStruct((2, N_NODES, HH), jnp.float32)
    else:
        out_spec = pl.BlockSpec((R_BLK, H), lambda i: (i, 0))
        out_shape = jax.ShapeDtypeStruct((N_NODES, H), jnp.float32)
    conv_specs = [
        pl.BlockSpec((1, R_BLK, HH), lambda i: (0, i, 0)),
        pl.BlockSpec((1, R_BLK, HH), lambda i: (1, i, 0)),
        pl.BlockSpec((R_BLK, 16), lambda i: (i, 0)),
        pl.BlockSpec((1, R_BLK, HH), lambda i: (0, i, 0)),
        pl.BlockSpec((1, R_BLK, HH), lambda i: (1, i, 0)),
        pl.BlockSpec((H, H), lambda i: (0, 0)),
        pl.BlockSpec((H, H), lambda i: (0, 0)),
        pl.BlockSpec((1, H), lambda i: (0, 0)),
    ]
    return pl.pallas_call(
        functools.partial(_layer_body, split),
        grid=(N_NODES // R_BLK,),
        in_specs=conv_specs + conv_specs,
        out_specs=[out_spec] * 2,
        out_shape=[out_shape] * 2,
    )(agg_i, agg_i, cnt_ui, xi, xi, wl_ui, wr_ui, b_ui.reshape(1, H),
      agg_u, agg_u, cnt_iu, xu, xu, wl_iu, wr_iu, b_iu.reshape(1, H))


def kernel(x_user, x_item, edge_index_ui, edge_index_iu,
           W_user, b_user, W_item, b_item,
           Wl_ui_0, Wr_ui_0, b_ui_0, Wl_iu_0, Wr_iu_0, b_iu_0,
           Wl_ui_1, Wr_ui_1, b_ui_1, Wl_iu_1, Wr_iu_1, b_iu_1):
    npad = EPAD - E
    pad_src = jnp.zeros((npad,), jnp.int32)
    pad_dst = (N_NODES + (jnp.arange(npad, dtype=jnp.int32) % 48))

    src_ui = jnp.concatenate([edge_index_ui[0], pad_src])
    dst_ui = jnp.concatenate([edge_index_ui[1], pad_dst])
    src_iu = jnp.concatenate([edge_index_iu[0], pad_src])
    dst_iu = jnp.concatenate([edge_index_iu[1], pad_dst])

    cnt2 = _counts(jnp.concatenate([dst_ui, dst_iu])).reshape(2, N_ACC, 16)
    cnt_ui = cnt2[0]
    cnt_iu = cnt2[1]

    xu, xi = _encode2(x_user, W_user, b_user, x_item, W_item, b_item)

    layers = [(Wl_ui_0, Wr_ui_0, b_ui_0, Wl_iu_0, Wr_iu_0, b_iu_0, True),
              (Wl_ui_1, Wr_ui_1, b_ui_1, Wl_iu_1, Wr_iu_1, b_iu_1, False)]
    for (Wl_ui, Wr_ui, b_ui, Wl_iu, Wr_iu, b_iu, split) in layers:
        agg_i = _segsum(xu.reshape(2 * N_NODES, HH), src_ui, dst_ui)
        agg_u = _segsum(xi.reshape(2 * N_NODES, HH), src_iu, dst_iu)
        new_xi, new_xu = _layer_dense(
            agg_i.reshape(2, N_ACC, HH), cnt_ui, xi, Wl_ui, Wr_ui, b_ui,
            agg_u.reshape(2, N_ACC, HH), cnt_iu, xu, Wl_iu, Wr_iu, b_iu,
            split)
        xu, xi = new_xu, new_xi
    return (xu, xi)
